# baseline (speedup 1.0000x reference)
import functools

import jax
import jax.numpy as jnp
from jax import lax
from jax.experimental import pallas as pl
from jax.experimental.pallas import tpu as pltpu
from jax.experimental.pallas import tpu_sc as plsc

BEV_H = 200
BEV_W = 200
HW = 32 * 88
NCAM = 6
NB = 2
BN = NB * NCAM
C = 64
NCELL = BEV_H * BEV_W


def _tc_scale_body(feat_ref, depth_ref, scaled_ref):
    feat = feat_ref[0]
    dep = depth_ref[0]
    conf = jnp.max(dep, axis=0, keepdims=True)
    scaled_ref[0] = feat * (conf * (1.0 / NCAM))


def _tc_scale(feat3, depth3):
    return pl.pallas_call(
        _tc_scale_body,
        grid=(BN,),
        in_specs=[
            pl.BlockSpec((1, C, HW), lambda i: (i, 0, 0)),
            pl.BlockSpec((1, C, HW), lambda i: (i, 0, 0)),
        ],
        out_specs=pl.BlockSpec((1, C, HW), lambda i: (i, 0, 0)),
        out_shape=jax.ShapeDtypeStruct((BN, C, HW), jnp.float32),
    )(feat3, depth3)


def _geometry_pack(I_inv, E_inv):
    sh = BEV_H / 100.0
    sw = BEV_W / 100.0
    V = jnp.array(
        [[0.0, -sw, BEV_W / 2.0], [-sh, 0.0, BEV_H / 2.0], [0.0, 0.0, 1.0]],
        dtype=jnp.float32,
    )
    h, w = 32, 88
    nb = I_inv.size // 9
    xs = jnp.linspace(0.0, w - 1, w)
    ys = jnp.linspace(0.0, h - 1, h)
    u, v = jnp.meshgrid(xs, ys, indexing="xy")
    pix = jnp.stack([u, v, jnp.ones_like(u)], axis=0).reshape(3, HW)
    I_inv_bn = I_inv.reshape(nb, 3, 3)
    E_inv_bn = E_inv.reshape(nb, 4, 4)
    cam = I_inv_bn @ pix
    cam = jnp.concatenate([cam, jnp.ones((nb, 1, HW), dtype=cam.dtype)], axis=1)
    d = jnp.transpose(E_inv_bn @ cam, (0, 2, 1))[..., :3]
    c = jnp.broadcast_to(E_inv_bn[:, :3, 3][:, None, :], (nb, HW, 3))
    dz = jnp.clip(d[..., 2], 1e-06, None)
    s = -c[..., 2] / dz
    xy_ego = c[..., :2] + d[..., :2] * s[..., None]
    homo = jnp.concatenate([xy_ego, jnp.ones_like(xy_ego[..., :1])], axis=-1)
    p = homo @ V.T
    bev_xy = p[..., :2] / jnp.clip(p[..., 2:], 1e-07, None)
    gx = bev_xy[..., 0] / (BEV_W - 1) * 2 - 1
    gy = bev_xy[..., 1] / (BEV_H - 1) * 2 - 1
    px = (gx + 1) * (BEV_W - 1) / 2
    py = (gy + 1) * (BEV_H - 1) / 2
    x0 = jnp.clip(jnp.floor(px), 0, BEV_W - 1)
    y0 = jnp.clip(jnp.floor(py), 0, BEV_H - 1)
    x1 = jnp.clip(x0 + 1, 0, BEV_W - 1)
    y1 = jnp.clip(y0 + 1, 0, BEV_H - 1)
    wa = (x1 - px) * (y1 - py)
    wb = (px - x0) * (y1 - py)
    wc = (x1 - px) * (py - y0)
    wd = (px - x0) * (py - y0)
    ia = y0 * BEV_W + x0
    sx = x1 - x0
    sy = (y1 - y0) * BEV_W
    return jnp.stack([ia, sx, sy, wa, wb, wc, wd], axis=1).reshape(nb, 7 * HW)


_NC = 2
_NS = 16
_NW = _NC * _NS
_CH_PER = C // _NW

_GROUPS = HW // 16


@functools.lru_cache(maxsize=1)
def _build_sc_splat():
  return functools.partial(
      pl.kernel,
      mesh=plsc.VectorSubcoreMesh(core_axis_name="c", subcore_axis_name="s"),
      out_type=jax.ShapeDtypeStruct((NB * C * NCELL,), jnp.float32),
      scratch_types=[
          pltpu.VMEM((_CH_PER * NCELL,), jnp.float32),
          pltpu.VMEM((2, _CH_PER, HW), jnp.float32),
          pltpu.VMEM((2, 7 * HW), jnp.float32),
          pltpu.SemaphoreType.DMA,
          pltpu.SemaphoreType.DMA,
          pltpu.SemaphoreType.DMA,
          pltpu.SemaphoreType.DMA,
      ],
      compiler_params=pltpu.CompilerParams(needs_layout_passes=False),
  )(_sc_splat_body)


def _sc_splat_body(
    feat_hbm, pack_hbm, out_hbm, acc, featb, packb, sem0, sem1, sem2, sem3
):
    wid = lax.axis_index("s") * _NC + lax.axis_index("c")
    ch0 = wid * _CH_PER
    sems = (sem0, sem1)
    fsems = (sem2, sem3)

    cp = pltpu.async_copy(pack_hbm.at[0], packb.at[0], sems[0])
    fcp0 = pltpu.async_copy(feat_hbm.at[0, ch0], featb.at[0, 0], fsems[0])
    fcp1 = pltpu.async_copy(feat_hbm.at[0, ch0 + 1], featb.at[0, 1], fsems[0])

    for b in range(NB):
        def zero_body(z, carry):
            for z8 in range(8):
                acc[pl.ds((z * 8 + z8) * 16, 16)] = jnp.zeros((16,), jnp.float32)
            return carry

        lax.fori_loop(0, (_CH_PER * NCELL) // 128, zero_body, 0)

        for cam in range(NCAM):
            bn = b * NCAM + cam
            cur = bn % 2
            fcp0.wait()
            fcp1.wait()
            cp.wait()
            if bn + 1 < BN:
                cp = pltpu.async_copy(
                    pack_hbm.at[bn + 1], packb.at[1 - cur], sems[1 - cur]
                )
                fcp0 = pltpu.async_copy(
                    feat_hbm.at[bn + 1, ch0], featb.at[1 - cur, 0], fsems[1 - cur]
                )
                fcp1 = pltpu.async_copy(
                    feat_hbm.at[bn + 1, ch0 + 1], featb.at[1 - cur, 1],
                    fsems[1 - cur],
                )

            def group_body(g, carry):
                base = g * 16
                f0 = featb[cur, 0, pl.ds(base, 16)]
                f1 = featb[cur, 1, pl.ds(base, 16)]
                ia = packb[cur, pl.ds(base, 16)].astype(jnp.int32)
                sx = packb[cur, pl.ds(HW + base, 16)].astype(jnp.int32)
                sy = packb[cur, pl.ds(2 * HW + base, 16)].astype(jnp.int32)
                iv = (ia, ia + sx, ia + sy, ia + sx + sy)
                wv = tuple(
                    packb[cur, pl.ds((3 + corner) * HW + base, 16)]
                    for corner in range(4)
                )
                uniform = jnp.max(ia) == jnp.min(ia)
                last = lax.iota(jnp.int32, 16) == 15

                @pl.when(uniform)
                def _():
                    for corner in range(4):
                        plsc.addupdate_scatter(
                            acc, [iv[corner]], plsc.cumsum(f0 * wv[corner]),
                            mask=last,
                        )
                        plsc.addupdate_scatter(
                            acc, [iv[corner] + NCELL],
                            plsc.cumsum(f1 * wv[corner]), mask=last,
                        )

                @pl.when(jnp.logical_not(uniform))
                def _():
                    for corner in range(4):
                        plsc.addupdate_scatter(acc, [iv[corner]], f0 * wv[corner])
                        plsc.addupdate_scatter(
                            acc, [iv[corner] + NCELL], f1 * wv[corner]
                        )

                return carry

            lax.fori_loop(0, _GROUPS, group_body, 0)

        base_out = (b * C + ch0) * NCELL
        pltpu.sync_copy(acc.at[pl.ds(0, NCELL)], out_hbm.at[pl.ds(base_out, NCELL)])
        pltpu.sync_copy(
            acc.at[pl.ds(NCELL, NCELL)],
            out_hbm.at[pl.ds(base_out + NCELL, NCELL)],
        )


def kernel(feat_bn, depth_prob, I_inv, E_inv, feat_hw, n):
    del feat_hw, n
    feat3 = feat_bn.reshape(BN, C, HW)
    depth3 = depth_prob.reshape(BN, C, HW)
    pack = _geometry_pack(I_inv, E_inv)
    scaled = _tc_scale(feat3, depth3)
    bev = _build_sc_splat()(scaled, pack)
    return bev.reshape(NB, C, BEV_H, BEV_W)


# speedup vs baseline: 1.1992x; 1.1992x over previous
import functools

import jax
import jax.numpy as jnp
from jax import lax
from jax.experimental import pallas as pl
from jax.experimental.pallas import tpu as pltpu
from jax.experimental.pallas import tpu_sc as plsc

BEV_H = 200
BEV_W = 200
HW = 32 * 88
NCAM = 6
NB = 2
BN = NB * NCAM
C = 64
NCELL = BEV_H * BEV_W


def _tc_scale_body(feat_ref, depth_ref, scaled_ref):
    feat = feat_ref[0]
    dep = depth_ref[0]
    conf = jnp.max(dep, axis=0, keepdims=True)
    scaled_ref[0] = feat * (conf * (1.0 / NCAM))


def _tc_scale(feat3, depth3):
    return pl.pallas_call(
        _tc_scale_body,
        grid=(BN,),
        in_specs=[
            pl.BlockSpec((1, C, HW), lambda i: (i, 0, 0)),
            pl.BlockSpec((1, C, HW), lambda i: (i, 0, 0)),
        ],
        out_specs=pl.BlockSpec((1, C, HW), lambda i: (i, 0, 0)),
        out_shape=jax.ShapeDtypeStruct((BN, C, HW), jnp.float32),
    )(feat3, depth3)


def _geometry_pack(I_inv, E_inv):
    sh = BEV_H / 100.0
    sw = BEV_W / 100.0
    V = jnp.array(
        [[0.0, -sw, BEV_W / 2.0], [-sh, 0.0, BEV_H / 2.0], [0.0, 0.0, 1.0]],
        dtype=jnp.float32,
    )
    h, w = 32, 88
    nb = I_inv.size // 9
    xs = jnp.linspace(0.0, w - 1, w)
    ys = jnp.linspace(0.0, h - 1, h)
    u, v = jnp.meshgrid(xs, ys, indexing="xy")
    pix = jnp.stack([u, v, jnp.ones_like(u)], axis=0).reshape(3, HW)
    I_inv_bn = I_inv.reshape(nb, 3, 3)
    E_inv_bn = E_inv.reshape(nb, 4, 4)
    cam = I_inv_bn @ pix
    cam = jnp.concatenate([cam, jnp.ones((nb, 1, HW), dtype=cam.dtype)], axis=1)
    d = jnp.transpose(E_inv_bn @ cam, (0, 2, 1))[..., :3]
    c = jnp.broadcast_to(E_inv_bn[:, :3, 3][:, None, :], (nb, HW, 3))
    dz = jnp.clip(d[..., 2], 1e-06, None)
    s = -c[..., 2] / dz
    xy_ego = c[..., :2] + d[..., :2] * s[..., None]
    homo = jnp.concatenate([xy_ego, jnp.ones_like(xy_ego[..., :1])], axis=-1)
    p = homo @ V.T
    bev_xy = p[..., :2] / jnp.clip(p[..., 2:], 1e-07, None)
    gx = bev_xy[..., 0] / (BEV_W - 1) * 2 - 1
    gy = bev_xy[..., 1] / (BEV_H - 1) * 2 - 1
    px = (gx + 1) * (BEV_W - 1) / 2
    py = (gy + 1) * (BEV_H - 1) / 2
    x0 = jnp.clip(jnp.floor(px), 0, BEV_W - 1)
    y0 = jnp.clip(jnp.floor(py), 0, BEV_H - 1)
    x1 = jnp.clip(x0 + 1, 0, BEV_W - 1)
    y1 = jnp.clip(y0 + 1, 0, BEV_H - 1)
    wa = (x1 - px) * (y1 - py)
    wb = (px - x0) * (y1 - py)
    wc = (x1 - px) * (py - y0)
    wd = (px - x0) * (py - y0)
    ia = y0 * BEV_W + x0
    sx = x1 - x0
    sy = (y1 - y0) * BEV_W
    return jnp.stack([ia, sx, sy, wa, wb, wc, wd], axis=1).reshape(nb, 7 * HW)


_NC = 2
_NS = 16
_NW = _NC * _NS
_CH_PER = C // _NW

_GROUPS = HW // 16


@functools.lru_cache(maxsize=1)
def _build_sc_splat():
  return functools.partial(
      pl.kernel,
      mesh=plsc.VectorSubcoreMesh(core_axis_name="c", subcore_axis_name="s"),
      out_type=jax.ShapeDtypeStruct((NB * C * NCELL,), jnp.float32),
      scratch_types=[
          pltpu.VMEM((_CH_PER * NCELL,), jnp.float32),
          pltpu.VMEM((2, _CH_PER, HW), jnp.float32),
          pltpu.VMEM((2, 7 * HW), jnp.float32),
          pltpu.SemaphoreType.DMA,
          pltpu.SemaphoreType.DMA,
          pltpu.SemaphoreType.DMA,
          pltpu.SemaphoreType.DMA,
      ],
      compiler_params=pltpu.CompilerParams(needs_layout_passes=False),
  )(_sc_splat_body)


def _sc_splat_body(
    feat_hbm, pack_hbm, out_hbm, acc, featb, packb, sem0, sem1, sem2, sem3
):
    wid = lax.axis_index("s") * _NC + lax.axis_index("c")
    ch0 = wid * _CH_PER
    sems = (sem0, sem1)
    fsems = (sem2, sem3)

    cp = pltpu.async_copy(pack_hbm.at[0], packb.at[0], sems[0])
    fcp0 = pltpu.async_copy(feat_hbm.at[0, ch0], featb.at[0, 0], fsems[0])
    fcp1 = pltpu.async_copy(feat_hbm.at[0, ch0 + 1], featb.at[0, 1], fsems[0])

    for b in range(NB):
        def zero_body(z, carry):
            for z8 in range(8):
                acc[pl.ds((z * 8 + z8) * 16, 16)] = jnp.zeros((16,), jnp.float32)
            return carry

        lax.fori_loop(0, (_CH_PER * NCELL) // 128, zero_body, 0)

        for cam in range(NCAM):
            bn = b * NCAM + cam
            cur = bn % 2
            fcp0.wait()
            fcp1.wait()
            cp.wait()
            if bn + 1 < BN:
                cp = pltpu.async_copy(
                    pack_hbm.at[bn + 1], packb.at[1 - cur], sems[1 - cur]
                )
                fcp0 = pltpu.async_copy(
                    feat_hbm.at[bn + 1, ch0], featb.at[1 - cur, 0], fsems[1 - cur]
                )
                fcp1 = pltpu.async_copy(
                    feat_hbm.at[bn + 1, ch0 + 1], featb.at[1 - cur, 1],
                    fsems[1 - cur],
                )

            @plsc.parallel_loop(0, _GROUPS, unroll=2)
            def group_body(g):
                base = g * 16
                f0 = featb[cur, 0, pl.ds(base, 16)]
                f1 = featb[cur, 1, pl.ds(base, 16)]
                ia = packb[cur, pl.ds(base, 16)].astype(jnp.int32)
                sx = packb[cur, pl.ds(HW + base, 16)].astype(jnp.int32)
                sy = packb[cur, pl.ds(2 * HW + base, 16)].astype(jnp.int32)
                iv = (ia, ia + sx, ia + sy, ia + sx + sy)
                wv = tuple(
                    packb[cur, pl.ds((3 + corner) * HW + base, 16)]
                    for corner in range(4)
                )
                uniform = jnp.max(ia) == jnp.min(ia)
                last = lax.iota(jnp.int32, 16) == 15

                @pl.when(uniform)
                def _():
                    for corner in range(4):
                        plsc.addupdate_scatter(
                            acc, [iv[corner]], plsc.cumsum(f0 * wv[corner]),
                            mask=last,
                        )
                        plsc.addupdate_scatter(
                            acc, [iv[corner] + NCELL],
                            plsc.cumsum(f1 * wv[corner]), mask=last,
                        )

                @pl.when(jnp.logical_not(uniform))
                def _():
                    for corner in range(4):
                        plsc.addupdate_scatter(acc, [iv[corner]], f0 * wv[corner])
                        plsc.addupdate_scatter(
                            acc, [iv[corner] + NCELL], f1 * wv[corner]
                        )


        base_out = (b * C + ch0) * NCELL
        pltpu.sync_copy(acc.at[pl.ds(0, NCELL)], out_hbm.at[pl.ds(base_out, NCELL)])
        pltpu.sync_copy(
            acc.at[pl.ds(NCELL, NCELL)],
            out_hbm.at[pl.ds(base_out + NCELL, NCELL)],
        )


def kernel(feat_bn, depth_prob, I_inv, E_inv, feat_hw, n):
    del feat_hw, n
    feat3 = feat_bn.reshape(BN, C, HW)
    depth3 = depth_prob.reshape(BN, C, HW)
    pack = _geometry_pack(I_inv, E_inv)
    scaled = _tc_scale(feat3, depth3)
    bev = _build_sc_splat()(scaled, pack)
    return bev.reshape(NB, C, BEV_H, BEV_W)
